# baseline (speedup 1.0000x reference)
import functools

import jax
import jax.numpy as jnp
from jax import lax
from jax.experimental import pallas as pl
from jax.experimental.pallas import tpu as pltpu
from jax.experimental.pallas import tpu_sc as plsc

F32 = jnp.float32
I32 = jnp.int32

NP = 50176
EP = 802816
HALF = 25088
NPW = 1568
SC = 32
SCH0 = 43
SCH1 = 55
EB = 1792
ES = 14
ECH = 28

_mesh = plsc.VectorSubcoreMesh(core_axis_name="c", subcore_axis_name="s")


@functools.partial(
    pl.kernel,
    out_type=(
        jax.ShapeDtypeStruct((200, 64), F32),
        jax.ShapeDtypeStruct((NP, 64), F32),
    ),
    mesh=_mesh,
    compiler_params=pltpu.CompilerParams(use_tc_tiling_on_sc=False),
    scratch_types=[
        pltpu.VMEM((3, 640), I32),
        pltpu.VMEM((3, 640, 64), F32),
        pltpu.VMEM((32, 64), F32),
        pltpu.SemaphoreType.DMA,
        pltpu.SemaphoreType.DMA,
        pltpu.SemaphoreType.DMA,
        pltpu.SemaphoreType.DMA,
        pltpu.SemaphoreType.DMA,
        pltpu.SemaphoreType.DMA,
    ],
)
def _sc_strings(emb_hbm, tsl_hbm, strf_hbm, tsrows_hbm, ssum_hbm,
                sidx_v, srows_v, sout_v, sem, sg0, sg1, sg2, si1, si2):
    cc = lax.axis_index("c")
    ss_ = lax.axis_index("s")
    wid = ss_ * 2 + cc
    sg = (sg0, sg1, sg2)
    si = (sem, si1, si2)
    nch = jnp.where(cc == 0, SCH0, SCH1)
    ntrip = jnp.where(cc == 0, (SCH0 - 1) // 3, (SCH1 - 1) // 3)

    @pl.when(wid < 25)
    def _():
        pltpu.sync_copy(tsl_hbm.at[pl.ds(wid * 8, 8)],
                        sidx_v.at[0].at[pl.ds(0, 8)])
        pltpu.async_copy(emb_hbm.at[sidx_v.at[0].at[pl.ds(0, 8)]],
                         sout_v.at[pl.ds(0, 8)], sem).wait()
        pltpu.sync_copy(sout_v.at[pl.ds(0, 8)],
                        tsrows_hbm.at[pl.ds(wid * 8, 8)])

    nbase = jnp.where(cc == 0, ss_ * (SCH0 * SC),
                      16 * SCH0 * SC + ss_ * (SCH1 * SC))

    def fire_idx(k, b):
        pltpu.async_copy(strf_hbm.at[pl.ds((nbase + k * SC) * 20, 640)],
                         sidx_v.at[b], si[b])

    def wait_idx(k, b):
        pltpu.make_async_copy(strf_hbm.at[pl.ds((nbase + k * SC) * 20, 640)],
                              sidx_v.at[b], si[b]).wait()

    def fire_g(b):
        for q in range(5):
            pltpu.async_copy(
                emb_hbm.at[sidx_v.at[b].at[pl.ds(q * 128, 128)]],
                srows_v.at[b].at[pl.ds(q * 128, 128)], sg[b])

    def drain(b):
        for q in range(5):
            pltpu.make_async_copy(
                emb_hbm.at[sidx_v.at[b].at[pl.ds(q * 128, 128)]],
                srows_v.at[b].at[pl.ds(q * 128, 128)], sg[b]).wait()

    def reduce(k, b):
        def node4(i, c2):
            r0 = i * 80
            n0 = i * 4
            for dn in range(4):
                for c4 in range(4):
                    sl = pl.ds(c4 * 16, 16)
                    acc = srows_v[b, r0 + dn * 20, sl]
                    for j in range(1, 20):
                        acc = acc + srows_v[b, r0 + dn * 20 + j, sl]
                    sout_v[n0 + dn, sl] = acc
            return c2

        lax.fori_loop(0, SC // 4, node4, 0)
        pltpu.sync_copy(sout_v, ssum_hbm.at[pl.ds(nbase + k * SC, SC)])

    pltpu.sync_copy(strf_hbm.at[pl.ds(nbase * 20, 640)], sidx_v.at[0])
    fire_g(0)
    fire_idx(1, 1)

    def triple(jj, carry):
        for b in range(3):
            k = jj * 3 + b

            @pl.when(k <= nch - 3)
            def _():
                fire_idx(k + 2, (b + 2) % 3)

            wait_idx(k + 1, (b + 1) % 3)
            fire_g((b + 1) % 3)
            drain(b)
            reduce(k, b)
        return carry

    lax.fori_loop(0, ntrip, triple, 0)
    drain(0)
    reduce(nch - 1, 0)


@functools.partial(
    pl.kernel,
    out_type=jax.ShapeDtypeStruct((NP, 64), F32),
    mesh=_mesh,
    compiler_params=pltpu.CompilerParams(use_tc_tiling_on_sc=False,
                                         needs_layout_passes=False),
    scratch_types=[
        pltpu.VMEM_SHARED((HALF + 16, 64), F32),
        pltpu.VMEM((2, EB + 16), I32),
        pltpu.VMEM((2, EB), I32),
        pltpu.VMEM((EB + 16,), I32),
        pltpu.VMEM((2, ES, 128), I32),
        pltpu.VMEM((2, 128, 64), F32),
        pltpu.SemaphoreType.DMA,
        pltpu.SemaphoreType.DMA,
        pltpu.SemaphoreType.DMA,
        pltpu.SemaphoreType.DMA,
    ],
)
def _sc_edges(hw_hbm, srcm_hbm, dstm_hbm, agg_hbm,
              agg_sp, srcv, dstv, cflat, clidx, rows_v, sg0, sg1, si0, si1):
    c = lax.axis_index("c")
    s = lax.axis_index("s")
    base_row = c * HALF
    dummy = HALF + s
    sg = (sg0, sg1)
    si = (si0, si1)
    iota = lax.broadcasted_iota(I32, (16,), 0)

    pltpu.sync_copy(hw_hbm.at[pl.ds(base_row + s * NPW, NPW)],
                    agg_sp.at[pl.ds(s * NPW, NPW)])
    plsc.subcore_barrier()

    def fire_idx(j, b):
        eb = s * 50176 + j * EB
        pltpu.async_copy(srcm_hbm.at[pl.ds(eb, EB)], srcv.at[b].at[pl.ds(0, EB)], si[b])
        pltpu.async_copy(dstm_hbm.at[pl.ds(eb, EB)], dstv.at[b], si[b])

    def wait_idx(j, b):
        eb = s * 50176 + j * EB
        pltpu.make_async_copy(srcm_hbm.at[pl.ds(eb, EB)],
                              srcv.at[b].at[pl.ds(0, EB)], si[b]).wait()
        pltpu.make_async_copy(dstm_hbm.at[pl.ds(eb, EB)], dstv.at[b],
                              si[b]).wait()

    def fire_g(b, q, sub, m):
        @pl.when(sub * 128 < m)
        def _():
            pltpu.async_copy(
                hw_hbm.at[srcv.at[b].at[pl.ds(sub * 128, 128)]],
                rows_v.at[q], sg[q])

    def wait_g(b, q, sub, m):
        @pl.when(sub * 128 < m)
        def _():
            pltpu.make_async_copy(
                hw_hbm.at[srcv.at[b].at[pl.ds(sub * 128, 128)]],
                rows_v.at[q], sg[q]).wait()

    def compact(b):
        def crow(r, off):
            for u in range(8):
                pos = r * 128 + u * 16
                d16 = dstv[b, pl.ds(pos, 16)]
                s16 = srcv[b, pl.ds(pos, 16)]
                l = d16 - base_row
                ok = (l >= 0) & (l < HALF)
                csum = plsc.cumsum(jnp.where(ok, 1, 0))
                cpos = jnp.where(ok, off + csum - 1, EB)
                plsc.store_scatter(srcv.at[b], [cpos], s16)
                plsc.store_scatter(cflat, [cpos], l)
                off = off + jnp.sum(jnp.where(ok, 1, 0))
            return off

        m = lax.fori_loop(0, ES, crow, 0)

        def cpy(r, c2):
            for u in range(8):
                pos = r * 128 + u * 16
                v = cflat[pl.ds(pos, 16)]
                clidx[b, r, pl.ds(u * 16, 16)] = jnp.where(
                    iota + pos < m, v, dummy)
            return c2

        lax.fori_loop(0, ES, cpy, 0)
        return m

    def block_body(j, b, mb, has_next):
        mn = 0
        if has_next:
            fire_idx(j + 1, 1 - b)
            wait_idx(j + 1, 1 - b)
            mn = compact(1 - b)
        for sub in range(ES):
            q = sub % 2
            wait_g(b, q, sub, mb)

            @pl.when(sub * 128 < mb)
            def _():
                pltpu.sync_copy(rows_v.at[q],
                                agg_sp.at[clidx.at[b].at[sub]], add=True)

            if sub + 2 < ES:
                fire_g(b, q, sub + 2, mb)
        if has_next:
            fire_g(1 - b, 0, 0, mn)
            fire_g(1 - b, 1, 1, mn)
        return mn

    pltpu.sync_copy(srcm_hbm.at[pl.ds(s * 50176, EB)],
                    srcv.at[0].at[pl.ds(0, EB)])
    pltpu.sync_copy(dstm_hbm.at[pl.ds(s * 50176, EB)], dstv.at[0])
    m0 = compact(0)
    fire_g(0, 0, 0, m0)
    fire_g(0, 1, 1, m0)

    def pair(jj, me):
        j0 = jj * 2
        mo = block_body(j0, 0, me, True)
        return block_body(j0 + 1, 1, mo, True)

    m26 = lax.fori_loop(0, ECH // 2 - 1, pair, m0)
    m27 = block_body(ECH - 2, 0, m26, True)
    block_body(ECH - 1, 1, m27, False)
    plsc.subcore_barrier()
    pltpu.sync_copy(agg_sp.at[pl.ds(s * NPW, NPW)],
                    agg_hbm.at[pl.ds(base_row + s * NPW, NPW)])


@functools.partial(
    pl.kernel,
    out_type=jax.ShapeDtypeStruct((1024, 64), F32),
    mesh=_mesh,
    compiler_params=pltpu.CompilerParams(use_tc_tiling_on_sc=False),
    scratch_types=[
        pltpu.VMEM((32,), I32),
        pltpu.VMEM((32, 64), F32),
        pltpu.VMEM((32, 64), F32),
        pltpu.VMEM((32, 64), F32),
        pltpu.SemaphoreType.DMA,
    ],
)
def _sc_readout(agg_hbm, h_hbm, ridx_hbm, out_hbm, iv, av, hv, ov, sem):
    wid = lax.axis_index("s") * 2 + lax.axis_index("c")
    pltpu.sync_copy(ridx_hbm.at[pl.ds(wid * 32, 32)], iv)
    pltpu.async_copy(agg_hbm.at[iv], av, sem).wait()
    pltpu.async_copy(h_hbm.at[iv], hv, sem).wait()

    def node(n, carry):
        for c4 in range(4):
            sl = pl.ds(c4 * 16, 16)
            ov[n, sl] = jnp.maximum(av[n, sl] + hv[n, sl], 0.0)
        return carry

    lax.fori_loop(0, 32, node, 0)
    pltpu.sync_copy(ov, out_hbm.at[pl.ds(wid * 32, 32)])


TB = 1792


def _tc_bottleneck(t_ref, ssum_ref, num_ref, tsr_ref, wka_ref, wn2_ref,
                   bn2_ref, wbnb_ref, wbnc_ref, wb1_ref, h0_ref, hw1_ref):
    keoa = jnp.dot(
        jnp.tanh(jnp.dot(tsr_ref[...], wka_ref[0:64, :],
                         preferred_element_type=F32)),
        wka_ref[64:128, :], preferred_element_type=F32)
    t0 = t_ref[:, 0]
    t1 = t_ref[:, 1]
    io = lax.broadcasted_iota(I32, (TB, 200), 1)
    ke0 = jnp.dot((t0[:, None] == io).astype(F32), keoa,
                  preferred_element_type=F32)
    ke1 = jnp.dot((t1[:, None] == io).astype(F32), keoa,
                  preferred_element_type=F32)
    kec = jnp.concatenate([ke0, ke1], axis=1)
    ne = jnp.tanh(
        jnp.dot(num_ref[...], wn2_ref[...], preferred_element_type=F32)
        + bn2_ref[0, :])
    x = (kec
         + jnp.dot(ssum_ref[...] * (1.0 / 20.0), wbnb_ref[...],
                   preferred_element_type=F32)
         + jnp.dot(ne, wbnc_ref[...], preferred_element_type=F32))
    h0 = jnp.tanh(x)
    h0_ref[...] = h0
    hw1_ref[...] = jnp.dot(h0, wb1_ref[...], preferred_element_type=F32)


def _tc_layer(agg_ref, h_ref, w_ref, h1_ref, hw2_ref):
    h1 = jnp.maximum(agg_ref[...] + h_ref[...], 0.0)
    h1_ref[...] = h1
    hw2_ref[...] = jnp.dot(h1, w_ref[...], preferred_element_type=F32)


def kernel(type_string, types, strings, numbers, edge_index, readout_idx,
           emb_table, W_key, W_num, b_num, W_bn, W_body1, W_body2):
    n = types.shape[0]
    e = edge_index.shape[1]

    tsl = type_string[:, -1].astype(I32)
    types2 = jnp.pad(types.astype(I32), (0, NP - n)).reshape(NP // 2, 2)
    strf = jnp.pad(strings.astype(I32), ((0, NP - n), (0, 0))).reshape(-1)
    num2 = jnp.pad(numbers, ((0, NP - n), (0, 0))).reshape(NP // 2, 16)
    src_e = edge_index[0].astype(I32)
    dst_e = edge_index[1].astype(I32)
    ridx = readout_idx.astype(I32)
    emb = emb_table.astype(F32)

    z64 = jnp.zeros((64, 64), F32)
    bd = lambda W: jnp.concatenate(
        [jnp.concatenate([W, z64], axis=1),
         jnp.concatenate([z64, W], axis=1)], axis=0)
    wka = jnp.concatenate([W_key, W_bn[0:64]], axis=0)
    wbnb2 = bd(W_bn[64:128])
    wbnc2 = bd(W_bn[128:192])
    wb1d = bd(W_body1)
    wb2d = bd(W_body2)
    z8 = jnp.zeros((8, 64), F32)
    wn2 = jnp.concatenate(
        [jnp.concatenate([W_num, z8], axis=1),
         jnp.concatenate([z8, W_num], axis=1)], axis=0)
    bn2 = jnp.concatenate([b_num, b_num]).reshape(1, 128).astype(F32)

    ts_rows, ssum = _sc_strings(emb, tsl, strf)
    ssum2 = ssum.reshape(NP // 2, 128)

    src_p = jnp.pad(src_e, (0, EP - e))
    dst_p = jnp.pad(dst_e, (0, EP - e), constant_values=2 ** 22)

    blk = lambda shp: pl.BlockSpec(shp, lambda i: (0,) * len(shp))
    row2 = lambda m: pl.BlockSpec((TB, m), lambda i: (i, 0))
    h0, hw1 = pl.pallas_call(
        _tc_bottleneck,
        grid=(NP // 2 // TB,),
        in_specs=[
            row2(2), row2(128), row2(16),
            blk((200, 64)), blk((128, 64)), blk((16, 128)), blk((1, 128)),
            blk((128, 128)), blk((128, 128)), blk((128, 128)),
        ],
        out_specs=[row2(128), row2(128)],
        out_shape=[
            jax.ShapeDtypeStruct((NP // 2, 128), F32),
            jax.ShapeDtypeStruct((NP // 2, 128), F32),
        ],
    )(types2, ssum2, num2, ts_rows, wka, wn2, bn2, wbnb2, wbnc2, wb1d)

    agg1 = _sc_edges(hw1.reshape(NP, 64), src_p, dst_p)

    h1, hw2 = pl.pallas_call(
        _tc_layer,
        grid=(NP // 2 // TB,),
        in_specs=[row2(128), row2(128), blk((128, 128))],
        out_specs=[row2(128), row2(128)],
        out_shape=[
            jax.ShapeDtypeStruct((NP // 2, 128), F32),
            jax.ShapeDtypeStruct((NP // 2, 128), F32),
        ],
    )(agg1.reshape(NP // 2, 128), h0, wb2d)

    agg2 = _sc_edges(hw2.reshape(NP, 64), src_p, dst_p)

    return _sc_readout(agg2, h1.reshape(NP, 64), ridx)


# speedup vs baseline: 1.0264x; 1.0264x over previous
import functools

import jax
import jax.numpy as jnp
from jax import lax
from jax.experimental import pallas as pl
from jax.experimental.pallas import tpu as pltpu
from jax.experimental.pallas import tpu_sc as plsc

F32 = jnp.float32
I32 = jnp.int32

NP = 50176
EP = 802816
HALF = 25088
NPW = 1568
SC = 32
SCH0 = 55
SCH1 = 43
EB = 1792
ES = 14
ECH = 28

_mesh = plsc.VectorSubcoreMesh(core_axis_name="c", subcore_axis_name="s")


@functools.partial(
    pl.kernel,
    out_type=(
        jax.ShapeDtypeStruct((200, 64), F32),
        jax.ShapeDtypeStruct((NP, 64), F32),
    ),
    mesh=_mesh,
    compiler_params=pltpu.CompilerParams(use_tc_tiling_on_sc=False),
    scratch_types=[
        pltpu.VMEM((3, 640), I32),
        pltpu.VMEM((3, 640, 64), F32),
        pltpu.VMEM((32, 64), F32),
        pltpu.SemaphoreType.DMA,
        pltpu.SemaphoreType.DMA,
        pltpu.SemaphoreType.DMA,
        pltpu.SemaphoreType.DMA,
        pltpu.SemaphoreType.DMA,
        pltpu.SemaphoreType.DMA,
    ],
)
def _sc_strings(emb_hbm, tsl_hbm, strf_hbm, tsrows_hbm, ssum_hbm,
                sidx_v, srows_v, sout_v, sem, sg0, sg1, sg2, si1, si2):
    cc = lax.axis_index("c")
    ss_ = lax.axis_index("s")
    wid = ss_ * 2 + cc
    sg = (sg0, sg1, sg2)
    si = (sem, si1, si2)
    nch = jnp.where(cc == 0, SCH0, SCH1)
    ntrip = jnp.where(cc == 0, (SCH0 - 1) // 3, (SCH1 - 1) // 3)

    @pl.when(wid < 25)
    def _():
        pltpu.sync_copy(tsl_hbm.at[pl.ds(wid * 8, 8)],
                        sidx_v.at[0].at[pl.ds(0, 8)])
        pltpu.async_copy(emb_hbm.at[sidx_v.at[0].at[pl.ds(0, 8)]],
                         sout_v.at[pl.ds(0, 8)], sem).wait()
        pltpu.sync_copy(sout_v.at[pl.ds(0, 8)],
                        tsrows_hbm.at[pl.ds(wid * 8, 8)])

    nbase = jnp.where(cc == 0, ss_ * (SCH0 * SC),
                      16 * SCH0 * SC + ss_ * (SCH1 * SC))

    def fire_idx(k, b):
        pltpu.async_copy(strf_hbm.at[pl.ds((nbase + k * SC) * 20, 640)],
                         sidx_v.at[b], si[b])

    def wait_idx(k, b):
        pltpu.make_async_copy(strf_hbm.at[pl.ds((nbase + k * SC) * 20, 640)],
                              sidx_v.at[b], si[b]).wait()

    def fire_g(b):
        for q in range(5):
            pltpu.async_copy(
                emb_hbm.at[sidx_v.at[b].at[pl.ds(q * 128, 128)]],
                srows_v.at[b].at[pl.ds(q * 128, 128)], sg[b])

    def drain(b):
        for q in range(5):
            pltpu.make_async_copy(
                emb_hbm.at[sidx_v.at[b].at[pl.ds(q * 128, 128)]],
                srows_v.at[b].at[pl.ds(q * 128, 128)], sg[b]).wait()

    def reduce(k, b):
        def node4(i, c2):
            r0 = i * 80
            n0 = i * 4
            for dn in range(4):
                for c4 in range(4):
                    sl = pl.ds(c4 * 16, 16)
                    acc = srows_v[b, r0 + dn * 20, sl]
                    for j in range(1, 20):
                        acc = acc + srows_v[b, r0 + dn * 20 + j, sl]
                    sout_v[n0 + dn, sl] = acc
            return c2

        lax.fori_loop(0, SC // 4, node4, 0)
        pltpu.sync_copy(sout_v, ssum_hbm.at[pl.ds(nbase + k * SC, SC)])

    pltpu.sync_copy(strf_hbm.at[pl.ds(nbase * 20, 640)], sidx_v.at[0])
    fire_g(0)
    fire_idx(1, 1)

    def triple(jj, carry):
        for b in range(3):
            k = jj * 3 + b

            @pl.when(k <= nch - 3)
            def _():
                fire_idx(k + 2, (b + 2) % 3)

            wait_idx(k + 1, (b + 1) % 3)
            fire_g((b + 1) % 3)
            drain(b)
            reduce(k, b)
        return carry

    lax.fori_loop(0, ntrip, triple, 0)
    drain(0)
    reduce(nch - 1, 0)


@functools.partial(
    pl.kernel,
    out_type=jax.ShapeDtypeStruct((NP, 64), F32),
    mesh=_mesh,
    compiler_params=pltpu.CompilerParams(use_tc_tiling_on_sc=False,
                                         needs_layout_passes=False),
    scratch_types=[
        pltpu.VMEM_SHARED((HALF + 16, 64), F32),
        pltpu.VMEM((2, EB + 16), I32),
        pltpu.VMEM((2, EB), I32),
        pltpu.VMEM((EB + 16,), I32),
        pltpu.VMEM((2, ES, 128), I32),
        pltpu.VMEM((2, 128, 64), F32),
        pltpu.SemaphoreType.DMA,
        pltpu.SemaphoreType.DMA,
        pltpu.SemaphoreType.DMA,
        pltpu.SemaphoreType.DMA,
    ],
)
def _sc_edges(hw_hbm, srcm_hbm, dstm_hbm, agg_hbm,
              agg_sp, srcv, dstv, cflat, clidx, rows_v, sg0, sg1, si0, si1):
    c = lax.axis_index("c")
    s = lax.axis_index("s")
    base_row = c * HALF
    dummy = HALF + s
    sg = (sg0, sg1)
    si = (si0, si1)
    iota = lax.broadcasted_iota(I32, (16,), 0)

    pltpu.sync_copy(hw_hbm.at[pl.ds(base_row + s * NPW, NPW)],
                    agg_sp.at[pl.ds(s * NPW, NPW)])
    plsc.subcore_barrier()

    def fire_idx(j, b):
        eb = s * 50176 + j * EB
        pltpu.async_copy(srcm_hbm.at[pl.ds(eb, EB)], srcv.at[b].at[pl.ds(0, EB)], si[b])
        pltpu.async_copy(dstm_hbm.at[pl.ds(eb, EB)], dstv.at[b], si[b])

    def wait_idx(j, b):
        eb = s * 50176 + j * EB
        pltpu.make_async_copy(srcm_hbm.at[pl.ds(eb, EB)],
                              srcv.at[b].at[pl.ds(0, EB)], si[b]).wait()
        pltpu.make_async_copy(dstm_hbm.at[pl.ds(eb, EB)], dstv.at[b],
                              si[b]).wait()

    def fire_g(b, q, sub, m):
        @pl.when(sub * 128 < m)
        def _():
            pltpu.async_copy(
                hw_hbm.at[srcv.at[b].at[pl.ds(sub * 128, 128)]],
                rows_v.at[q], sg[q])

    def wait_g(b, q, sub, m):
        @pl.when(sub * 128 < m)
        def _():
            pltpu.make_async_copy(
                hw_hbm.at[srcv.at[b].at[pl.ds(sub * 128, 128)]],
                rows_v.at[q], sg[q]).wait()

    def compact(b):
        def crow(r, off):
            for u in range(8):
                pos = r * 128 + u * 16
                d16 = dstv[b, pl.ds(pos, 16)]
                s16 = srcv[b, pl.ds(pos, 16)]
                l = d16 - base_row
                ok = (l >= 0) & (l < HALF)
                csum = plsc.cumsum(jnp.where(ok, 1, 0))
                cpos = jnp.where(ok, off + csum - 1, EB)
                plsc.store_scatter(srcv.at[b], [cpos], s16)
                plsc.store_scatter(cflat, [cpos], l)
                off = off + jnp.sum(jnp.where(ok, 1, 0))
            return off

        m = lax.fori_loop(0, ES, crow, 0)

        def cpy(r, c2):
            for u in range(8):
                pos = r * 128 + u * 16
                v = cflat[pl.ds(pos, 16)]
                clidx[b, r, pl.ds(u * 16, 16)] = jnp.where(
                    iota + pos < m, v, dummy)
            return c2

        lax.fori_loop(0, ES, cpy, 0)
        return m

    def block_body(j, b, mb, has_next):
        mn = 0
        if has_next:
            fire_idx(j + 1, 1 - b)
            wait_idx(j + 1, 1 - b)
            mn = compact(1 - b)
        for sub in range(ES):
            q = sub % 2
            wait_g(b, q, sub, mb)

            @pl.when(sub * 128 < mb)
            def _():
                pltpu.sync_copy(rows_v.at[q],
                                agg_sp.at[clidx.at[b].at[sub]], add=True)

            if sub + 2 < ES:
                fire_g(b, q, sub + 2, mb)
        if has_next:
            fire_g(1 - b, 0, 0, mn)
            fire_g(1 - b, 1, 1, mn)
        return mn

    pltpu.sync_copy(srcm_hbm.at[pl.ds(s * 50176, EB)],
                    srcv.at[0].at[pl.ds(0, EB)])
    pltpu.sync_copy(dstm_hbm.at[pl.ds(s * 50176, EB)], dstv.at[0])
    m0 = compact(0)
    fire_g(0, 0, 0, m0)
    fire_g(0, 1, 1, m0)

    def pair(jj, me):
        j0 = jj * 2
        mo = block_body(j0, 0, me, True)
        return block_body(j0 + 1, 1, mo, True)

    m26 = lax.fori_loop(0, ECH // 2 - 1, pair, m0)
    m27 = block_body(ECH - 2, 0, m26, True)
    block_body(ECH - 1, 1, m27, False)
    plsc.subcore_barrier()
    pltpu.sync_copy(agg_sp.at[pl.ds(s * NPW, NPW)],
                    agg_hbm.at[pl.ds(base_row + s * NPW, NPW)])


@functools.partial(
    pl.kernel,
    out_type=jax.ShapeDtypeStruct((1024, 64), F32),
    mesh=_mesh,
    compiler_params=pltpu.CompilerParams(use_tc_tiling_on_sc=False),
    scratch_types=[
        pltpu.VMEM((32,), I32),
        pltpu.VMEM((32, 64), F32),
        pltpu.VMEM((32, 64), F32),
        pltpu.VMEM((32, 64), F32),
        pltpu.SemaphoreType.DMA,
    ],
)
def _sc_readout(agg_hbm, h_hbm, ridx_hbm, out_hbm, iv, av, hv, ov, sem):
    wid = lax.axis_index("s") * 2 + lax.axis_index("c")
    pltpu.sync_copy(ridx_hbm.at[pl.ds(wid * 32, 32)], iv)
    pltpu.async_copy(agg_hbm.at[iv], av, sem).wait()
    pltpu.async_copy(h_hbm.at[iv], hv, sem).wait()

    def node(n, carry):
        for c4 in range(4):
            sl = pl.ds(c4 * 16, 16)
            ov[n, sl] = jnp.maximum(av[n, sl] + hv[n, sl], 0.0)
        return carry

    lax.fori_loop(0, 32, node, 0)
    pltpu.sync_copy(ov, out_hbm.at[pl.ds(wid * 32, 32)])


TB = 1792


def _tc_bottleneck(t_ref, ssum_ref, num_ref, tsr_ref, wka_ref, wn2_ref,
                   bn2_ref, wbnb_ref, wbnc_ref, wb1_ref, h0_ref, hw1_ref):
    keoa = jnp.dot(
        jnp.tanh(jnp.dot(tsr_ref[...], wka_ref[0:64, :],
                         preferred_element_type=F32)),
        wka_ref[64:128, :], preferred_element_type=F32)
    t0 = t_ref[:, 0]
    t1 = t_ref[:, 1]
    io = lax.broadcasted_iota(I32, (TB, 200), 1)
    ke0 = jnp.dot((t0[:, None] == io).astype(F32), keoa,
                  preferred_element_type=F32)
    ke1 = jnp.dot((t1[:, None] == io).astype(F32), keoa,
                  preferred_element_type=F32)
    kec = jnp.concatenate([ke0, ke1], axis=1)
    ne = jnp.tanh(
        jnp.dot(num_ref[...], wn2_ref[...], preferred_element_type=F32)
        + bn2_ref[0, :])
    x = (kec
         + jnp.dot(ssum_ref[...] * (1.0 / 20.0), wbnb_ref[...],
                   preferred_element_type=F32)
         + jnp.dot(ne, wbnc_ref[...], preferred_element_type=F32))
    h0 = jnp.tanh(x)
    h0_ref[...] = h0
    hw1_ref[...] = jnp.dot(h0, wb1_ref[...], preferred_element_type=F32)


def _tc_layer(agg_ref, h_ref, w_ref, h1_ref, hw2_ref):
    h1 = jnp.maximum(agg_ref[...] + h_ref[...], 0.0)
    h1_ref[...] = h1
    hw2_ref[...] = jnp.dot(h1, w_ref[...], preferred_element_type=F32)


def kernel(type_string, types, strings, numbers, edge_index, readout_idx,
           emb_table, W_key, W_num, b_num, W_bn, W_body1, W_body2):
    n = types.shape[0]
    e = edge_index.shape[1]

    tsl = type_string[:, -1].astype(I32)
    types2 = jnp.pad(types.astype(I32), (0, NP - n)).reshape(NP // 2, 2)
    strf = jnp.pad(strings.astype(I32), ((0, NP - n), (0, 0))).reshape(-1)
    num2 = jnp.pad(numbers, ((0, NP - n), (0, 0))).reshape(NP // 2, 16)
    src_e = edge_index[0].astype(I32)
    dst_e = edge_index[1].astype(I32)
    ridx = readout_idx.astype(I32)
    emb = emb_table.astype(F32)

    z64 = jnp.zeros((64, 64), F32)
    bd = lambda W: jnp.concatenate(
        [jnp.concatenate([W, z64], axis=1),
         jnp.concatenate([z64, W], axis=1)], axis=0)
    wka = jnp.concatenate([W_key, W_bn[0:64]], axis=0)
    wbnb2 = bd(W_bn[64:128])
    wbnc2 = bd(W_bn[128:192])
    wb1d = bd(W_body1)
    wb2d = bd(W_body2)
    z8 = jnp.zeros((8, 64), F32)
    wn2 = jnp.concatenate(
        [jnp.concatenate([W_num, z8], axis=1),
         jnp.concatenate([z8, W_num], axis=1)], axis=0)
    bn2 = jnp.concatenate([b_num, b_num]).reshape(1, 128).astype(F32)

    ts_rows, ssum = _sc_strings(emb, tsl, strf)
    ssum2 = ssum.reshape(NP // 2, 128)

    src_p = jnp.pad(src_e, (0, EP - e))
    dst_p = jnp.pad(dst_e, (0, EP - e), constant_values=2 ** 22)

    blk = lambda shp: pl.BlockSpec(shp, lambda i: (0,) * len(shp))
    row2 = lambda m: pl.BlockSpec((TB, m), lambda i: (i, 0))
    h0, hw1 = pl.pallas_call(
        _tc_bottleneck,
        grid=(NP // 2 // TB,),
        in_specs=[
            row2(2), row2(128), row2(16),
            blk((200, 64)), blk((128, 64)), blk((16, 128)), blk((1, 128)),
            blk((128, 128)), blk((128, 128)), blk((128, 128)),
        ],
        out_specs=[row2(128), row2(128)],
        out_shape=[
            jax.ShapeDtypeStruct((NP // 2, 128), F32),
            jax.ShapeDtypeStruct((NP // 2, 128), F32),
        ],
    )(types2, ssum2, num2, ts_rows, wka, wn2, bn2, wbnb2, wbnc2, wb1d)

    agg1 = _sc_edges(hw1.reshape(NP, 64), src_p, dst_p)

    h1, hw2 = pl.pallas_call(
        _tc_layer,
        grid=(NP // 2 // TB,),
        in_specs=[row2(128), row2(128), blk((128, 128))],
        out_specs=[row2(128), row2(128)],
        out_shape=[
            jax.ShapeDtypeStruct((NP // 2, 128), F32),
            jax.ShapeDtypeStruct((NP // 2, 128), F32),
        ],
    )(agg1.reshape(NP // 2, 128), h0, wb2d)

    agg2 = _sc_edges(hw2.reshape(NP, 64), src_p, dst_p)

    return _sc_readout(agg2, h1.reshape(NP, 64), ridx)
